# Initial kernel scaffold; baseline (speedup 1.0000x reference)
import jax
import jax.numpy as jnp
from jax.experimental import pallas as pl


def kernel(x, weight):
    raise NotImplementedError("write your pallas kernel here")


# speedup vs baseline: 1.4923x; 1.4923x over previous
import functools

import jax
import jax.numpy as jnp
from jax import lax
from jax.experimental import pallas as pl
from jax.experimental.pallas import tpu as pltpu
from jax.experimental.pallas import tpu_sc as plsc

VOCAB = 1000000
EMB_DIM = 32
BATCH = 4096
HIST = 200

NUM_CORES = 2
NUM_SUBCORES = 16
NUM_WORKERS = NUM_CORES * NUM_SUBCORES

ROWS_TOTAL = BATCH * HIST
ROWS_PER_WORKER = ROWS_TOTAL // NUM_WORKERS

IDX_MINOR = 128
GATHERS_PER_CHUNK = 8
CHUNK_ROWS = IDX_MINOR * GATHERS_PER_CHUNK
NUM_CHUNKS = ROWS_PER_WORKER // CHUNK_ROWS
IDX_ROWS_PER_WORKER = ROWS_PER_WORKER // IDX_MINOR


def _gather_kernel(weight_hbm, idx_hbm, out_hbm, idx_v, rows_v, sem_g, sem_o):
    c = lax.axis_index("c")
    s = lax.axis_index("s")
    wid = s * NUM_CORES + c

    pltpu.sync_copy(idx_hbm.at[pl.ds(wid * IDX_ROWS_PER_WORKER,
                                     IDX_ROWS_PER_WORKER)], idx_v)

    out_base = wid * ROWS_PER_WORKER

    def chunk_body(g, carry):
        buf = lax.rem(g, 2)
        waits = []
        for b in range(GATHERS_PER_CHUNK):
            cp = pltpu.async_copy(
                weight_hbm.at[idx_v.at[g * GATHERS_PER_CHUNK + b]],
                rows_v.at[buf, pl.ds(b * IDX_MINOR, IDX_MINOR)],
                sem_g)
            waits.append(cp)
        for cp in waits:
            cp.wait()
        out_cp = pltpu.async_copy(
            rows_v.at[buf],
            out_hbm.at[pl.ds(out_base + g * CHUNK_ROWS, CHUNK_ROWS)],
            sem_o)
        @pl.when(g > 0)
        def _():
            pltpu.make_async_copy(
                rows_v.at[1 - buf],
                out_hbm.at[pl.ds(out_base + (g - 1) * CHUNK_ROWS, CHUNK_ROWS)],
                sem_o).wait()
        return carry

    lax.fori_loop(0, NUM_CHUNKS, chunk_body, 0)
    last = NUM_CHUNKS - 1
    pltpu.make_async_copy(
        rows_v.at[last % 2],
        out_hbm.at[pl.ds(out_base + last * CHUNK_ROWS, CHUNK_ROWS)],
        sem_o).wait()


@jax.jit
def kernel(x, weight):
    idx2d = x.reshape(ROWS_TOTAL // IDX_MINOR, IDX_MINOR)
    mesh = plsc.VectorSubcoreMesh(core_axis_name="c", subcore_axis_name="s")
    out = pl.kernel(
        _gather_kernel,
        mesh=mesh,
        out_type=jax.ShapeDtypeStruct((ROWS_TOTAL, EMB_DIM), jnp.float32),
        scratch_types=[
            pltpu.VMEM((IDX_ROWS_PER_WORKER, IDX_MINOR), jnp.int32),
            pltpu.VMEM((2, CHUNK_ROWS, EMB_DIM), jnp.float32),
            pltpu.SemaphoreType.DMA,
            pltpu.SemaphoreType.DMA,
        ],
        compiler_params=pltpu.CompilerParams(use_tc_tiling_on_sc=False),
    )(weight, idx2d)
    return out.reshape(BATCH, HIST, EMB_DIM)
